# baseline (speedup 1.0000x reference)
import functools

import jax
import jax.numpy as jnp
from jax import lax
from jax.experimental import pallas as pl
from jax.experimental.pallas import tpu as pltpu
from jax.experimental.pallas import tpu_sc as plsc

VOCAB = 100000
CTX = 20
EMB = 256
HID = 512
B = 1024

ROWS = B * CTX
NC, NS = 2, 16
NW = NC * NS
ROWS_PER_W = ROWS // NW
CHUNK = 320
NCHUNK = ROWS_PER_W // CHUNK

VB = 2048
NVB = -(-VOCAB // VB)


def _sc_gather(emb, idx):
    mesh = plsc.VectorSubcoreMesh(core_axis_name="c", subcore_axis_name="s")

    @functools.partial(
        pl.kernel,
        out_type=jax.ShapeDtypeStruct((ROWS, EMB), jnp.float32),
        mesh=mesh,
        scratch_types=[
            pltpu.VMEM((ROWS_PER_W,), jnp.int32),
            pltpu.VMEM((CHUNK, EMB), jnp.float32),
            pltpu.SemaphoreType.DMA,
        ],
    )
    def k(emb_hbm, idx_hbm, out_hbm, idx_v, rows_v, sem):
        wid = lax.axis_index("s") * NC + lax.axis_index("c")
        base = wid * ROWS_PER_W
        pltpu.sync_copy(idx_hbm.at[pl.ds(base, ROWS_PER_W)], idx_v)
        for c in range(NCHUNK):
            pltpu.async_copy(
                emb_hbm.at[idx_v.at[pl.ds(c * CHUNK, CHUNK)]], rows_v, sem
            ).wait()
            pltpu.sync_copy(rows_v, out_hbm.at[pl.ds(base + c * CHUNK, CHUNK)])

    return k(emb, idx)


def _mlp_body(x_ref, w1_ref, b1_ref, w2_ref, b2_ref, out_ref, h_ref):
    @pl.when(pl.program_id(0) == 0)
    def _():
        h = lax.dot_general(
            x_ref[...], w1_ref[...], (((1,), (1,)), ((), ())),
            preferred_element_type=jnp.float32,
        )
        h_ref[...] = jnp.maximum(h + b1_ref[...], 0.0).astype(jnp.bfloat16)

    out_ref[...] = lax.dot_general(
        h_ref[...], w2_ref[...].astype(jnp.bfloat16), (((1,), (1,)), ((), ())),
        preferred_element_type=jnp.float32,
    ) + b2_ref[...]


def _mlp(x, W1, b1, W2, b2):
    return pl.pallas_call(
        _mlp_body,
        grid=(NVB,),
        in_specs=[
            pl.BlockSpec((B, CTX * EMB), lambda j: (0, 0)),
            pl.BlockSpec((HID, CTX * EMB), lambda j: (0, 0)),
            pl.BlockSpec((1, HID), lambda j: (0, 0)),
            pl.BlockSpec((VB, HID), lambda j: (j, 0)),
            pl.BlockSpec((1, VB), lambda j: (0, j)),
        ],
        out_specs=pl.BlockSpec((B, VB), lambda j: (0, j)),
        out_shape=jax.ShapeDtypeStruct((B, VOCAB), jnp.float32),
        scratch_shapes=[pltpu.VMEM((B, HID), jnp.bfloat16)],
    )(x, W1, b1.reshape(1, HID), W2, b2.reshape(1, VOCAB))


def kernel(inputs, emb, W1, b1, W2, b2):
    idx = inputs.reshape(-1).astype(jnp.int32)
    x = _sc_gather(emb, idx).reshape(B, CTX * EMB)
    return _mlp(x, W1, b1, W2, b2)


# speedup vs baseline: 1.0070x; 1.0070x over previous
import functools

import jax
import jax.numpy as jnp
from jax import lax
from jax.experimental import pallas as pl
from jax.experimental.pallas import tpu as pltpu
from jax.experimental.pallas import tpu_sc as plsc

VOCAB = 100000
CTX = 20
EMB = 256
HID = 512
B = 1024

ROWS = B * CTX
NC, NS = 2, 16
NW = NC * NS
ROWS_PER_W = ROWS // NW
CHUNK = 320
NCHUNK = ROWS_PER_W // CHUNK

VB = 4096
NVB = -(-VOCAB // VB)


def _sc_gather(emb, idx):
    mesh = plsc.VectorSubcoreMesh(core_axis_name="c", subcore_axis_name="s")

    @functools.partial(
        pl.kernel,
        out_type=jax.ShapeDtypeStruct((ROWS, EMB), jnp.float32),
        mesh=mesh,
        scratch_types=[
            pltpu.VMEM((ROWS_PER_W,), jnp.int32),
            pltpu.VMEM((CHUNK, EMB), jnp.float32),
            pltpu.SemaphoreType.DMA,
        ],
    )
    def k(emb_hbm, idx_hbm, out_hbm, idx_v, rows_v, sem):
        wid = lax.axis_index("s") * NC + lax.axis_index("c")
        base = wid * ROWS_PER_W
        pltpu.sync_copy(idx_hbm.at[pl.ds(base, ROWS_PER_W)], idx_v)
        for c in range(NCHUNK):
            pltpu.async_copy(
                emb_hbm.at[idx_v.at[pl.ds(c * CHUNK, CHUNK)]], rows_v, sem
            ).wait()
            pltpu.sync_copy(rows_v, out_hbm.at[pl.ds(base + c * CHUNK, CHUNK)])

    return k(emb, idx)


def _l1_body(x_ref, w1_ref, b1_ref, h_ref):
    h = lax.dot_general(
        x_ref[...], w1_ref[...], (((1,), (1,)), ((), ())),
        preferred_element_type=jnp.float32,
    )
    h_ref[...] = jnp.maximum(h + b1_ref[...], 0.0).astype(jnp.bfloat16)


def _l1(x, W1, b1):
    return pl.pallas_call(
        _l1_body,
        out_shape=jax.ShapeDtypeStruct((B, HID), jnp.bfloat16),
    )(x, W1, b1.reshape(1, HID))


def _l2_body(h_ref, w2_ref, b2_ref, out_ref):
    out_ref[...] = lax.dot_general(
        h_ref[...], w2_ref[...].astype(jnp.bfloat16), (((1,), (1,)), ((), ())),
        preferred_element_type=jnp.float32,
    ) + b2_ref[...]


def _l2(h, W2, b2):
    return pl.pallas_call(
        _l2_body,
        grid=(NVB,),
        in_specs=[
            pl.BlockSpec((B, HID), lambda j: (0, 0)),
            pl.BlockSpec((VB, HID), lambda j: (j, 0)),
            pl.BlockSpec((1, VB), lambda j: (0, j)),
        ],
        out_specs=pl.BlockSpec((B, VB), lambda j: (0, j)),
        out_shape=jax.ShapeDtypeStruct((B, VOCAB), jnp.float32),
    )(h, W2, b2.reshape(1, VOCAB))


def kernel(inputs, emb, W1, b1, W2, b2):
    idx = inputs.reshape(-1).astype(jnp.int32)
    x = _sc_gather(emb, idx).reshape(B, CTX * EMB)
    h = _l1(x, W1, b1)
    return _l2(h, W2, b2)


# speedup vs baseline: 2.2676x; 2.2519x over previous
import functools

import jax
import jax.numpy as jnp
from jax import lax
from jax.experimental import pallas as pl
from jax.experimental.pallas import tpu as pltpu
from jax.experimental.pallas import tpu_sc as plsc
from jax.experimental.layout import Format, Layout, with_layout_constraint

VOCAB = 100000
CTX = 20
EMB = 256
HID = 512
B = 1024

ROWS = B * CTX
NC, NS = 2, 16
NW = NC * NS
ROWS_PER_W = ROWS // NW
CHUNK = 320
NCHUNK = ROWS_PER_W // CHUNK

VB = 2048
NVB = -(-VOCAB // VB)


def _sc_gather(emb, idx):
    mesh = plsc.VectorSubcoreMesh(core_axis_name="c", subcore_axis_name="s")

    @functools.partial(
        pl.kernel,
        out_type=jax.ShapeDtypeStruct((ROWS, EMB), jnp.float32),
        mesh=mesh,
        scratch_types=[
            pltpu.VMEM((ROWS_PER_W,), jnp.int32),
            pltpu.VMEM((CHUNK, EMB), jnp.float32),
            pltpu.SemaphoreType.DMA,
        ],
    )
    def k(emb_hbm, idx_hbm, out_hbm, idx_v, rows_v, sem):
        wid = lax.axis_index("s") * NC + lax.axis_index("c")
        base = wid * ROWS_PER_W
        pltpu.sync_copy(idx_hbm.at[pl.ds(base, ROWS_PER_W)], idx_v)
        for c in range(NCHUNK):
            pltpu.async_copy(
                emb_hbm.at[idx_v.at[pl.ds(c * CHUNK, CHUNK)]], rows_v, sem
            ).wait()
            pltpu.sync_copy(rows_v, out_hbm.at[pl.ds(base + c * CHUNK, CHUNK)])

    return k(emb, idx)


def _l1_body(x_ref, w1_ref, b1_ref, h_ref):
    h = lax.dot_general(
        x_ref[...], w1_ref[...], (((1,), (1,)), ((), ())),
        preferred_element_type=jnp.float32,
    )
    h_ref[...] = jnp.maximum(h + b1_ref[...], 0.0).astype(jnp.bfloat16)


def _l1(x, W1, b1):
    return pl.pallas_call(
        _l1_body,
        out_shape=jax.ShapeDtypeStruct((B, HID), jnp.bfloat16),
    )(x, W1, b1.reshape(1, HID))


def _l2_body(h_ref, w2_ref, b2_ref, out_ref):
    acc = lax.dot_general(
        w2_ref[...].astype(jnp.bfloat16), h_ref[...], (((1,), (1,)), ((), ())),
        preferred_element_type=jnp.float32,
    )
    out_ref[...] = acc + jnp.transpose(b2_ref[...])


def _l2(h, W2, b2):
    return pl.pallas_call(
        _l2_body,
        grid=(NVB,),
        in_specs=[
            pl.BlockSpec((B, HID), lambda j: (0, 0)),
            pl.BlockSpec((VB, HID), lambda j: (j, 0)),
            pl.BlockSpec((1, VB), lambda j: (0, j)),
        ],
        out_specs=pl.BlockSpec((VB, B), lambda j: (j, 0)),
        out_shape=jax.ShapeDtypeStruct((VOCAB, B), jnp.float32),
    )(h, W2, b2.reshape(1, VOCAB))


def kernel(inputs, emb, W1, b1, W2, b2):
    idx = inputs.reshape(-1).astype(jnp.int32)
    x = _sc_gather(emb, idx).reshape(B, CTX * EMB)
    h = _l1(x, W1, b1)
    out = _l2(h, W2, b2).T
    return with_layout_constraint(out, Layout((1, 0)))


# speedup vs baseline: 2.4717x; 1.0900x over previous
import functools

import jax
import jax.numpy as jnp
from jax import lax
from jax.experimental import pallas as pl
from jax.experimental.pallas import tpu as pltpu
from jax.experimental.pallas import tpu_sc as plsc
from jax.experimental.layout import Format, Layout, with_layout_constraint

VOCAB = 100000
CTX = 20
EMB = 256
HID = 512
B = 1024

ROWS = B * CTX
NC, NS = 2, 16
NW = NC * NS
ROWS_PER_W = ROWS // NW
CHUNK = 320
NCHUNK = ROWS_PER_W // CHUNK

VB = 2048
NVB = -(-VOCAB // VB)


def _sc_gather(emb, idx):
    mesh = plsc.VectorSubcoreMesh(core_axis_name="c", subcore_axis_name="s")

    @functools.partial(
        pl.kernel,
        out_type=jax.ShapeDtypeStruct((ROWS, EMB), jnp.float32),
        mesh=mesh,
        scratch_types=[
            pltpu.VMEM((ROWS_PER_W,), jnp.int32),
            pltpu.VMEM((CHUNK, EMB), jnp.float32),
            pltpu.SemaphoreType.DMA,
        ],
    )
    def k(emb_hbm, idx_hbm, out_hbm, idx_v, rows_v, sem):
        wid = lax.axis_index("s") * NC + lax.axis_index("c")
        base = wid * ROWS_PER_W
        pltpu.sync_copy(idx_hbm.at[pl.ds(base, ROWS_PER_W)], idx_v)
        for c in range(NCHUNK):
            pltpu.async_copy(
                emb_hbm.at[idx_v.at[pl.ds(c * CHUNK, CHUNK)]], rows_v, sem
            ).wait()
            pltpu.sync_copy(rows_v, out_hbm.at[pl.ds(base + c * CHUNK, CHUNK)])

    return k(emb, idx)


def _l1_body(x_ref, w1_ref, b1_ref, h_ref):
    acc = b1_ref[...].astype(jnp.float32)
    acc = jnp.broadcast_to(acc, (B, HID))
    for c in range(CTX):
        acc = acc + lax.dot_general(
            x_ref[c], w1_ref[:, c * EMB:(c + 1) * EMB], (((1,), (1,)), ((), ())),
            preferred_element_type=jnp.float32,
        )
    h_ref[...] = jnp.maximum(acc, 0.0).astype(jnp.bfloat16)


def _l1(x3, W1, b1):
    return pl.pallas_call(
        _l1_body,
        out_shape=jax.ShapeDtypeStruct((B, HID), jnp.bfloat16),
    )(x3, W1, b1.reshape(1, HID))


def _l2_body(h_ref, w2_ref, b2_ref, out_ref):
    acc = lax.dot_general(
        w2_ref[...].astype(jnp.bfloat16), h_ref[...], (((1,), (1,)), ((), ())),
        preferred_element_type=jnp.float32,
    )
    out_ref[...] = acc + jnp.transpose(b2_ref[...])


def _l2(h, W2, b2):
    return pl.pallas_call(
        _l2_body,
        grid=(NVB,),
        in_specs=[
            pl.BlockSpec((B, HID), lambda j: (0, 0)),
            pl.BlockSpec((VB, HID), lambda j: (j, 0)),
            pl.BlockSpec((1, VB), lambda j: (0, j)),
        ],
        out_specs=pl.BlockSpec((VB, B), lambda j: (j, 0)),
        out_shape=jax.ShapeDtypeStruct((VOCAB, B), jnp.float32),
    )(h, W2, b2.reshape(1, VOCAB))


def kernel(inputs, emb, W1, b1, W2, b2):
    idx = inputs.T.reshape(-1).astype(jnp.int32)
    x3 = _sc_gather(emb, idx).reshape(CTX, B, EMB)
    h = _l1(x3, W1, b1)
    out = _l2(h, W2, b2).T
    return with_layout_constraint(out, Layout((1, 0)))
